# baseline (speedup 1.0000x reference)
import functools

import jax
import jax.numpy as jnp
from jax import lax
from jax.experimental import pallas as pl
from jax.experimental.pallas import tpu as pltpu
from jax.experimental.pallas import tpu_sc as plsc

VOCAB = 100000
DIM = 64
BATCH = 1024
CTX = 200

NC = 2
NS = 16
NW = NC * NS
L = 16

N_PASS = DIM // NW
BLK = 64
NB = BATCH // BLK
CB = BLK // L


BH = BATCH // 2


def _sc_pool_t(tableT, idxT, batch_n):
  nb = batch_n // BLK

  @functools.partial(
      pl.kernel,
      out_type=jax.ShapeDtypeStruct((DIM, batch_n), jnp.float32),
      mesh=plsc.VectorSubcoreMesh(core_axis_name="c", subcore_axis_name="s"),
      compiler_params=pltpu.CompilerParams(
          use_tc_tiling_on_sc=False, needs_layout_passes=False),
      scratch_types=[
          pltpu.VMEM((VOCAB,), jnp.float32),
          pltpu.VMEM((2, CTX, BLK), jnp.int32),
          pltpu.VMEM((batch_n,), jnp.float32),
          pltpu.SemaphoreType.DMA,
          pltpu.SemaphoreType.DMA,
      ],
  )
  def sc_kernel(tableT_hbm, idxT_hbm, out_hbm, row_v, idx_v, pooled_v,
                sem0, sem1):
    wid = lax.axis_index("s") * NC + lax.axis_index("c")
    sems = (sem0, sem1)

    def issue(g, buf):
      pltpu.async_copy(
          idxT_hbm.at[:, pl.ds(g * BLK, BLK)], idx_v.at[buf], sems[buf])

    def drain(g, buf):
      pltpu.make_async_copy(
          idxT_hbm.at[:, pl.ds(g * BLK, BLK)], idx_v.at[buf], sems[buf]).wait()

    def block_accum(g, buf):
      def body_j(j, accs):
        out = list(accs)
        for c in range(CB):
          idxv = idx_v[buf, j, pl.ds(c * L, L)]
          out[c] = out[c] + plsc.load_gather(row_v, [idxv])
        return tuple(out)

      accs = lax.fori_loop(
          0, CTX, body_j,
          tuple(jnp.zeros((L,), jnp.float32) for _ in range(CB)),
          unroll=4)
      for c in range(CB):
        pooled_v[pl.ds(g * BLK + c * L, L)] = accs[c] * (1.0 / CTX)

    for p in range(N_PASS):
      d = wid + p * NW
      pltpu.sync_copy(tableT_hbm.at[d], row_v)
      issue(0, 0)

      def body_pair(i, carry):
        g0 = 2 * i
        drain(g0, 0)
        issue(g0 + 1, 1)
        block_accum(g0, 0)
        drain(g0 + 1, 1)

        @pl.when(g0 + 2 < nb)
        def _():
          issue(g0 + 2, 0)

        block_accum(g0 + 1, 1)
        return carry

      lax.fori_loop(0, nb // 2, body_pair, 0)
      pltpu.sync_copy(pooled_v, out_hbm.at[d])

  return sc_kernel(tableT, idxT)


V_TILE = 5120
V_GRID = (VOCAB + V_TILE - 1) // V_TILE


def _mm_body(wt_ref, xt_ref, b_ref, o_ref):
  ones = jnp.ones((1, BH), jnp.float32)
  o_ref[...] = lax.dot_general(
      wt_ref[...], xt_ref[...],
      dimension_numbers=(((0,), (0,)), ((), ())),
      preferred_element_type=jnp.float32) + lax.dot_general(
          b_ref[...], ones,
          dimension_numbers=(((0,), (0,)), ((), ())),
          preferred_element_type=jnp.float32)


def _mm_body2(wt_ref, xt_ref, b_ref, prev_ref, o_ref):
  del prev_ref
  _mm_body(wt_ref, xt_ref, b_ref, o_ref)


def _tc_project_half(pooledT_h, Wt, b2d, half, prev=None):
  common = dict(
      grid=(V_GRID,),
      out_specs=pl.BlockSpec((V_TILE, BH), lambda i: (i, half)),
      out_shape=jax.ShapeDtypeStruct((VOCAB, BATCH), jnp.float32),
  )
  in_specs = [
      pl.BlockSpec((DIM, V_TILE), lambda i: (0, i)),
      pl.BlockSpec((DIM, BH), lambda i: (0, 0)),
      pl.BlockSpec((1, V_TILE), lambda i: (0, i)),
  ]
  if prev is None:
    return pl.pallas_call(_mm_body, in_specs=in_specs, **common)(
        Wt, pooledT_h, b2d)
  in_specs.append(pl.BlockSpec(memory_space=pl.ANY))
  return pl.pallas_call(
      _mm_body2, in_specs=in_specs, input_output_aliases={3: 0}, **common)(
          Wt, pooledT_h, b2d, prev)


def kernel(emb_table, W, b, inputs):
  tableT = emb_table.T
  idxT = inputs.T
  Wt = W.T
  b2d = b.reshape(1, VOCAB)
  pooledT_0 = _sc_pool_t(tableT, idxT[:, :BH], BH)
  pooledT_1 = _sc_pool_t(tableT, idxT[:, BH:], BH)
  out_t = _tc_project_half(pooledT_0, Wt, b2d, 0)
  out_t = _tc_project_half(pooledT_1, Wt, b2d, 1, prev=out_t)
  return out_t.T


# speedup vs baseline: 1.1448x; 1.1448x over previous
import functools

import jax
import jax.numpy as jnp
from jax import lax
from jax.experimental import pallas as pl
from jax.experimental.pallas import tpu as pltpu
from jax.experimental.pallas import tpu_sc as plsc

VOCAB = 100000
DIM = 64
BATCH = 1024
CTX = 200

NC = 2
NS = 16
NW = NC * NS
L = 16

N_PASS = DIM // NW
BLK = 64
NB = BATCH // BLK
CB = BLK // L


def _sc_pool_t(tableT, idxT):
  @functools.partial(
      pl.kernel,
      out_type=jax.ShapeDtypeStruct((DIM, BATCH), jnp.float32),
      mesh=plsc.VectorSubcoreMesh(core_axis_name="c", subcore_axis_name="s"),
      compiler_params=pltpu.CompilerParams(
          use_tc_tiling_on_sc=False, needs_layout_passes=False),
      scratch_types=[
          pltpu.VMEM((VOCAB,), jnp.float32),
          pltpu.VMEM((2, CTX, BLK), jnp.int32),
          pltpu.VMEM((BATCH,), jnp.float32),
          pltpu.SemaphoreType.DMA,
          pltpu.SemaphoreType.DMA,
      ],
  )
  def sc_kernel(tableT_hbm, idxT_hbm, out_hbm, row_v, idx_v, pooled_v,
                sem0, sem1):
    wid = lax.axis_index("s") * NC + lax.axis_index("c")
    sems = (sem0, sem1)

    def issue(g, buf):
      pltpu.async_copy(
          idxT_hbm.at[:, pl.ds(g * BLK, BLK)], idx_v.at[buf], sems[buf])

    def drain(g, buf):
      pltpu.make_async_copy(
          idxT_hbm.at[:, pl.ds(g * BLK, BLK)], idx_v.at[buf], sems[buf]).wait()

    def block_accum(g, buf):
      def body_j(j, accs):
        out = list(accs)
        for c in range(CB):
          idxv = idx_v[buf, j, pl.ds(c * L, L)]
          out[c] = out[c] + plsc.load_gather(row_v, [idxv])
        return tuple(out)

      accs = lax.fori_loop(
          0, CTX, body_j,
          tuple(jnp.zeros((L,), jnp.float32) for _ in range(CB)),
          unroll=2)
      for c in range(CB):
        pooled_v[pl.ds(g * BLK + c * L, L)] = accs[c] * (1.0 / CTX)

    for p in range(N_PASS):
      d = wid + p * NW
      pltpu.sync_copy(tableT_hbm.at[d], row_v)
      issue(0, 0)

      def body_pair(i, carry):
        g0 = 2 * i
        drain(g0, 0)
        issue(g0 + 1, 1)
        block_accum(g0, 0)
        drain(g0 + 1, 1)

        @pl.when(g0 + 2 < NB)
        def _():
          issue(g0 + 2, 0)

        block_accum(g0 + 1, 1)
        return carry

      lax.fori_loop(0, NB // 2, body_pair, 0)
      pltpu.sync_copy(pooled_v, out_hbm.at[d])

  return sc_kernel(tableT, idxT)


V_TILE = 4096
V_GRID = (VOCAB + V_TILE - 1) // V_TILE


def _mm_body(wt_ref, xt_ref, b_ref, o_ref):
  ones = jnp.ones((1, BATCH), jnp.float32)
  o_ref[...] = lax.dot_general(
      wt_ref[...], xt_ref[...],
      dimension_numbers=(((0,), (0,)), ((), ())),
      preferred_element_type=jnp.float32) + lax.dot_general(
          b_ref[...], ones,
          dimension_numbers=(((0,), (0,)), ((), ())),
          preferred_element_type=jnp.float32)


def _tc_project(pooledT, Wt, b2d):
  return pl.pallas_call(
      _mm_body,
      grid=(V_GRID,),
      in_specs=[
          pl.BlockSpec((DIM, V_TILE), lambda i: (0, i)),
          pl.BlockSpec((DIM, BATCH), lambda i: (0, 0)),
          pl.BlockSpec((1, V_TILE), lambda i: (0, i)),
      ],
      out_specs=pl.BlockSpec((V_TILE, BATCH), lambda i: (i, 0)),
      out_shape=jax.ShapeDtypeStruct((VOCAB, BATCH), jnp.float32),
  )(Wt, pooledT, b2d)


def kernel(emb_table, W, b, inputs):
  pooledT = _sc_pool_t(emb_table.T, inputs.T)
  out_t = _tc_project(pooledT, W.T, b.reshape(1, VOCAB))
  return out_t.T
